# baseline (speedup 1.0000x reference)
import jax
import jax.numpy as jnp
from jax import lax
from jax.experimental import pallas as pl
from jax.experimental.pallas import tpu as pltpu
from jax.experimental.pallas import tpu_sc as plsc

_B = 4096
_D = 16
_ALPHA = 0.2
_NB = 10
_BR = 512
_NW = 32
_RPW = _B // _NW
_RBLK = 8
_NBLK = _RPW // _RBLK
_L = 16
_NEG = -1e30


def _score_block(a_ref, b_ref, out_ref):
    step = pl.program_id(0)
    a = a_ref[...]
    b = b_ref[...]
    hb = 0.5 * jnp.sum(b * b, axis=1)[None, :]
    ab = lax.dot_general(a, b, (((1,), (1,)), ((), ())),
                         preferred_element_type=jnp.float32)
    out_ref[...] = ab - hb
    li = lax.broadcasted_iota(jnp.int32, (_BR, _BR), 0)
    lj = lax.broadcasted_iota(jnp.int32, (_BR, _BR), 1)
    w = out_ref[:, pl.ds(step * _BR, _BR)]
    out_ref[:, pl.ds(step * _BR, _BR)] = jnp.where(li == lj, _NEG, w)


def _rsqrt_vec(x):
    u = plsc.bitcast(x, jnp.int32)
    y = plsc.bitcast(jnp.int32(0x5F3759DF) - lax.shift_right_logical(u, 1),
                     jnp.float32)
    for _ in range(3):
        y = y * (1.5 - 0.5 * x * y * y)
    return y


def _sc_top16(m_hbm, a_hbm, b_hbm, out_hbm, buf0, buf1, abuf, bbuf, acc_v,
              sem0, sem1, sem2):
    wid = lax.axis_index("s") * 2 + lax.axis_index("c")
    r0 = wid * _RPW
    bufs = (buf0, buf1)
    sems = (sem0, sem1)
    lane = lax.broadcasted_iota(jnp.int32, (_L,), 0)
    keep = lane >= (_L - _NB)
    acc = jnp.zeros((_L,), jnp.float32)
    copies = [None, None]
    copies[0] = pltpu.async_copy(m_hbm.at[pl.ds(r0, _RBLK)], buf0, sem0)
    ab_cp = pltpu.async_copy(a_hbm.at[pl.ds(r0, _RPW)], abuf, sem2)
    bb_cp = pltpu.async_copy(b_hbm.at[pl.ds(r0, _RPW)], bbuf, sem2)
    ab_cp.wait()
    bb_cp.wait()
    for blk in range(_NBLK):
        if blk + 1 < _NBLK:
            nxt = (blk + 1) % 2
            copies[nxt] = pltpu.async_copy(
                m_hbm.at[pl.ds(r0 + (blk + 1) * _RBLK, _RBLK)],
                bufs[nxt], sems[nxt])
        copies[blk % 2].wait()
        cur = bufs[blk % 2]

        def body(c, tops):
            new = []
            for r in range(_RBLK):
                g = cur[r, pl.ds(c * _L, _L)]
                g_desc = -lax.sort(-g)
                u = jnp.maximum(tops[r], g_desc)
                t_asc = lax.sort(u)
                new.append(t_asc)
            return tuple(new)

        tops = lax.fori_loop(
            0, _B // _L, body,
            tuple(jnp.full((_L,), _NEG, jnp.float32) for _ in range(_RBLK)))

        for r in range(_RBLK):
            av = abuf[blk * _RBLK + r, :]
            bv = bbuf[blk * _RBLK + r, :]
            df = av - bv
            pos2 = lax.broadcast(
                jnp.maximum(jnp.sum(df * df), 1e-12), (_L,))
            pos = pos2 * _rsqrt_vec(pos2)
            a2 = lax.broadcast(jnp.sum(av * av), (_L,))
            d2 = jnp.maximum(a2 - 2.0 * tops[r], 1e-12)
            d = d2 * _rsqrt_vec(d2)
            cost = jnp.maximum(pos - d + _ALPHA, 0.0)
            acc = acc + jnp.where(keep, cost, 0.0)
    acc_v[...] = acc * (1.0 / (_B * _NB))
    pltpu.sync_copy(acc_v, out_hbm.at[wid])


_sc_call = pl.kernel(
    _sc_top16,
    out_type=jax.ShapeDtypeStruct((_NW, _L), jnp.float32),
    mesh=plsc.VectorSubcoreMesh(core_axis_name="c", subcore_axis_name="s"),
    scratch_types=[
        pltpu.VMEM((_RBLK, _B), jnp.float32),
        pltpu.VMEM((_RBLK, _B), jnp.float32),
        pltpu.VMEM((_RPW, _D), jnp.float32),
        pltpu.VMEM((_RPW, _D), jnp.float32),
        pltpu.VMEM((_L,), jnp.float32),
        pltpu.SemaphoreType.DMA,
        pltpu.SemaphoreType.DMA,
        pltpu.SemaphoreType.DMA,
    ],
    compiler_params=pltpu.CompilerParams(needs_layout_passes=False),
)


def kernel(input1, input2, target, class1, class2):
    m = pl.pallas_call(
        _score_block,
        grid=(_B // _BR,),
        in_specs=[
            pl.BlockSpec((_BR, _D), lambda i: (i, 0)),
            pl.BlockSpec((_B, _D), lambda i: (0, 0)),
        ],
        out_specs=pl.BlockSpec((_BR, _B), lambda i: (i, 0)),
        out_shape=jax.ShapeDtypeStruct((_B, _B), jnp.float32),
    )(input1, input2)
    parts = _sc_call(m, input1, input2)
    return jnp.sum(parts)


# speedup vs baseline: 1.0555x; 1.0555x over previous
import jax
import jax.numpy as jnp
from jax import lax
from jax.experimental import pallas as pl
from jax.experimental.pallas import tpu as pltpu
from jax.experimental.pallas import tpu_sc as plsc

_B = 4096
_D = 16
_ALPHA = 0.2
_NB = 10
_BR = 512
_NW = 32
_RPW = _B // _NW
_RBLK = 8
_NBLK = _RPW // _RBLK
_L = 16
_NEG = -1e30


def _score_block(a_ref, b_ref, out_ref):
    step = pl.program_id(0)
    a = a_ref[...]
    b = b_ref[...]
    hb = 0.5 * jnp.sum(b * b, axis=1)[None, :]
    ab = lax.dot_general(a, b, (((1,), (1,)), ((), ())),
                         preferred_element_type=jnp.float32)
    out_ref[...] = ab - hb
    li = lax.broadcasted_iota(jnp.int32, (_BR, _BR), 0)
    lj = lax.broadcasted_iota(jnp.int32, (_BR, _BR), 1)
    w = out_ref[:, pl.ds(step * _BR, _BR)]
    out_ref[:, pl.ds(step * _BR, _BR)] = jnp.where(li == lj, _NEG, w)


def _rsqrt_vec(x):
    u = plsc.bitcast(x, jnp.int32)
    y = plsc.bitcast(jnp.int32(0x5F3759DF) - lax.shift_right_logical(u, 1),
                     jnp.float32)
    for _ in range(3):
        y = y * (1.5 - 0.5 * x * y * y)
    return y


def _sc_top16(m_hbm, a_hbm, b_hbm, out_hbm, buf0, buf1, abuf, bbuf, acc_v,
              sem0, sem1, sem2):
    wid = lax.axis_index("s") * 2 + lax.axis_index("c")
    r0 = wid * _RPW
    bufs = (buf0, buf1)
    sems = (sem0, sem1)
    lane = lax.broadcasted_iota(jnp.int32, (_L,), 0)
    keep = lane >= (_L - _NB)
    acc = jnp.zeros((_L,), jnp.float32)
    copies = [None, None]
    copies[0] = pltpu.async_copy(m_hbm.at[pl.ds(r0, _RBLK)], buf0, sem0)
    ab_cp = pltpu.async_copy(a_hbm.at[pl.ds(r0, _RPW)], abuf, sem2)
    bb_cp = pltpu.async_copy(b_hbm.at[pl.ds(r0, _RPW)], bbuf, sem2)
    ab_cp.wait()
    bb_cp.wait()
    for blk in range(_NBLK):
        if blk + 1 < _NBLK:
            nxt = (blk + 1) % 2
            copies[nxt] = pltpu.async_copy(
                m_hbm.at[pl.ds(r0 + (blk + 1) * _RBLK, _RBLK)],
                bufs[nxt], sems[nxt])
        copies[blk % 2].wait()
        cur = bufs[blk % 2]

        def body(c, tops):
            new = []
            for r in range(_RBLK):
                g = cur[r, pl.ds(c * _L, _L)]
                g_desc, _ = plsc.sort_key_val(g, g, descending=True)
                u = jnp.maximum(tops[r], g_desc)
                t_asc, _ = plsc.sort_key_val(u, u)
                new.append(t_asc)
            return tuple(new)

        tops = lax.fori_loop(
            0, _B // _L, body,
            tuple(jnp.full((_L,), _NEG, jnp.float32) for _ in range(_RBLK)))

        for r in range(_RBLK):
            av = abuf[blk * _RBLK + r, :]
            bv = bbuf[blk * _RBLK + r, :]
            df = av - bv
            pos2 = lax.broadcast(
                jnp.maximum(jnp.sum(df * df), 1e-12), (_L,))
            pos = pos2 * _rsqrt_vec(pos2)
            a2 = lax.broadcast(jnp.sum(av * av), (_L,))
            d2 = jnp.maximum(a2 - 2.0 * tops[r], 1e-12)
            d = d2 * _rsqrt_vec(d2)
            cost = jnp.maximum(pos - d + _ALPHA, 0.0)
            acc = acc + jnp.where(keep, cost, 0.0)
    acc_v[...] = acc * (1.0 / (_B * _NB))
    pltpu.sync_copy(acc_v, out_hbm.at[wid])


_sc_call = pl.kernel(
    _sc_top16,
    out_type=jax.ShapeDtypeStruct((_NW, _L), jnp.float32),
    mesh=plsc.VectorSubcoreMesh(core_axis_name="c", subcore_axis_name="s"),
    scratch_types=[
        pltpu.VMEM((_RBLK, _B), jnp.float32),
        pltpu.VMEM((_RBLK, _B), jnp.float32),
        pltpu.VMEM((_RPW, _D), jnp.float32),
        pltpu.VMEM((_RPW, _D), jnp.float32),
        pltpu.VMEM((_L,), jnp.float32),
        pltpu.SemaphoreType.DMA,
        pltpu.SemaphoreType.DMA,
        pltpu.SemaphoreType.DMA,
    ],
    compiler_params=pltpu.CompilerParams(needs_layout_passes=False),
)


def kernel(input1, input2, target, class1, class2):
    m = pl.pallas_call(
        _score_block,
        grid=(_B // _BR,),
        in_specs=[
            pl.BlockSpec((_BR, _D), lambda i: (i, 0)),
            pl.BlockSpec((_B, _D), lambda i: (0, 0)),
        ],
        out_specs=pl.BlockSpec((_BR, _B), lambda i: (i, 0)),
        out_shape=jax.ShapeDtypeStruct((_B, _B), jnp.float32),
    )(input1, input2)
    parts = _sc_call(m, input1, input2)
    return jnp.sum(parts)
